# baseline (speedup 1.0000x reference)
import functools

import jax
import jax.numpy as jnp
from jax import lax
from jax.experimental import pallas as pl
from jax.experimental.pallas import tpu as pltpu
from jax.experimental.pallas import tpu_sc as plsc

NC, NS = 2, 16
NW = NC * NS
B = 16384
D = 64
BPW = B // NW
NG = BPW // 16
KRING = 11
FDEPTH = 10
OCHUNK = 64
NOCH = BPW // OCHUNK

_mesh = plsc.VectorSubcoreMesh(core_axis_name="c", subcore_axis_name="s")


@functools.partial(
    pl.kernel,
    out_type=jax.ShapeDtypeStruct((B, 2 * D), jnp.float32),
    mesh=_mesh,
    scratch_types=[
        pltpu.VMEM((B,), jnp.int32),
        pltpu.VMEM((BPW,), jnp.int32),
        pltpu.VMEM((BPW,), jnp.int32),
        pltpu.VMEM((BPW,), jnp.int32),
        pltpu.VMEM((BPW + 16,), jnp.int32),
        pltpu.VMEM((NOCH, OCHUNK), jnp.int32),
        pltpu.VMEM((KRING, D, 128), jnp.float32),
        pltpu.VMEM((2, OCHUNK, 2 * D), jnp.float32),
        pltpu.SemaphoreType.DMA((KRING,)),
        pltpu.SemaphoreType.DMA((2,)),
    ],
    compiler_params=pltpu.CompilerParams(needs_layout_passes=False),
)
def _lookup(key_hbm, row_hbm, allids_hbm, tblt_hbm, out_hbm,
            allids_v, key_v, new_v, fc_v, runs_v, row_v, ring_v, big_v,
            gsems, osems):
    wid = lax.axis_index("s") * NC + lax.axis_index("c")

    pltpu.sync_copy(key_hbm.at[wid], key_v)
    pltpu.sync_copy(row_hbm.at[wid], row_v)
    pltpu.sync_copy(allids_hbm, allids_v)

    rows16 = lax.iota(jnp.int32, 16)
    zrow = jnp.zeros((16,), jnp.float32)

    def _pre(g, nf):
        g16 = g * 16
        slabv = key_v[pl.ds(g16, 16)] >> 14
        prevv = plsc.load_gather(
            key_v, [jnp.maximum(rows16 + g16 - 1, 0)]
        ) >> 14
        newv = jnp.where(g16 + rows16 == 0, 1,
                         (slabv != prevv).astype(jnp.int32))
        fcv = plsc.cumsum(newv) + nf
        new_v[pl.ds(g16, 16)] = newv
        fc_v[pl.ds(g16, 16)] = fcv
        plsc.store_scatter(runs_v, [fcv - 1], slabv)
        return fcv[15]

    nruns = lax.fori_loop(0, NG, _pre, 0)

    def _fire(sl, slot):
        col = pl.multiple_of(sl * 128, 128)
        pltpu.async_copy(
            tblt_hbm.at[:, pl.ds(col, 128)], ring_v.at[slot], gsems.at[slot]
        )

    runs0 = runs_v[pl.ds(0, 16)]
    for r in range(FDEPTH):
        @pl.when(r < nruns)
        def _():
            _fire(runs0[r], r % KRING)

    def _zero(b, carry):
        for c in range(D // 16):
            big_v[b // OCHUNK, b % OCHUNK, pl.ds(D + c * 16, 16)] = zrow
        return carry

    lax.fori_loop(0, 2 * OCHUNK, _zero, 0)

    def _step(g, want_prev):
        g16 = g * 16
        posv = row_v[g // 4, pl.ds((g % 4) * 16, 16)]
        idv = plsc.load_gather(allids_v, [posv])
        newv = new_v[pl.ds(g16, 16)]
        fcv = fc_v[pl.ds(g16, 16)]
        obank = (g // 4) % 2

        @pl.when(jnp.logical_and(g % 4 == 0, g >= 8))
        def _():
            pltpu.make_async_copy(
                big_v.at[obank], out_hbm.at[row_v.at[0]], osems.at[obank]
            ).wait()

        wp = want_prev
        for r in range(16):
            fci = fcv[r]
            slot = (fci - 1) % KRING

            want = jnp.minimum(fci + FDEPTH, nruns)
            fire_k = want - 1
            fslab = plsc.load_gather(runs_v, [jnp.full((16,), fire_k)])[0]

            @pl.when(want > wp)
            def _():
                _fire(fslab, fire_k % KRING)

            wp = want

            @pl.when(newv[r] == 1)
            def _():
                pltpu.make_async_copy(
                    tblt_hbm.at[:, pl.ds(0, 128)], ring_v.at[slot],
                    gsems.at[slot],
                ).wait()

            colv = jnp.full((16,), idv[r] & 127, jnp.int32)
            brow = (g % 4) * 16 + r
            for c in range(D // 16):
                vals = plsc.load_gather(ring_v.at[slot], [rows16 + c * 16, colv])
                big_v[obank, brow, pl.ds(c * 16, 16)] = vals

        @pl.when(g % 4 == 3)
        def _():
            pltpu.async_copy(
                big_v.at[obank],
                out_hbm.at[row_v.at[g // 4]],
                osems.at[obank],
            )

        return wp

    lax.fori_loop(0, NG, _step, jnp.minimum(FDEPTH, nruns))

    for obank in range(2):
        pltpu.make_async_copy(
            big_v.at[obank], out_hbm.at[row_v.at[0]], osems.at[obank]
        ).wait()


def kernel(entity_ids, entity_table):
    ids = entity_ids.astype(jnp.int32)
    pos = lax.iota(jnp.int32, B)
    key = ((ids >> 7) << 14) | pos
    key_s = lax.sort(key, is_stable=False)
    rows = (key_s & (B - 1)).reshape(NW, NOCH, OCHUNK)
    return _lookup(key_s.reshape(NW, BPW), rows, ids, entity_table.T)


# speedup vs baseline: 1.0251x; 1.0251x over previous
import functools

import jax
import jax.numpy as jnp
from jax import lax
from jax.experimental import pallas as pl
from jax.experimental.pallas import tpu as pltpu
from jax.experimental.pallas import tpu_sc as plsc

NC, NS = 2, 16
NW = NC * NS
B = 16384
D = 64
BPW = B // NW
NG = BPW // 16
KRING = 12
FDEPTH = 11
OCHUNK = 64
NOCH = BPW // OCHUNK

_mesh = plsc.VectorSubcoreMesh(core_axis_name="c", subcore_axis_name="s")


@functools.partial(
    pl.kernel,
    out_type=jax.ShapeDtypeStruct((B, 2 * D), jnp.float32),
    mesh=_mesh,
    scratch_types=[
        pltpu.VMEM((BPW,), jnp.int32),
        pltpu.VMEM((BPW,), jnp.int32),
        pltpu.VMEM((BPW,), jnp.int32),
        pltpu.VMEM((BPW + 16,), jnp.int32),
        pltpu.VMEM((NOCH, OCHUNK), jnp.int32),
        pltpu.VMEM((KRING, D, 128), jnp.float32),
        pltpu.VMEM((2, OCHUNK, 2 * D), jnp.float32),
        pltpu.SemaphoreType.DMA((KRING,)),
        pltpu.SemaphoreType.DMA((2,)),
    ],
    compiler_params=pltpu.CompilerParams(needs_layout_passes=False),
)
def _lookup(ids_hbm, row_hbm, tblt_hbm, out_hbm,
            ids_v, new_v, fc_v, runs_v, row_v, ring_v, big_v, gsems, osems):
    wid = lax.axis_index("s") * NC + lax.axis_index("c")

    pltpu.sync_copy(ids_hbm.at[wid], ids_v)
    pltpu.sync_copy(row_hbm.at[wid], row_v)

    rows16 = lax.iota(jnp.int32, 16)
    zrow = jnp.zeros((16,), jnp.float32)

    def _pre(g, nf):
        g16 = g * 16
        slabv = ids_v[pl.ds(g16, 16)] >> 7
        prevv = plsc.load_gather(
            ids_v, [jnp.maximum(rows16 + g16 - 1, 0)]
        ) >> 7
        newv = jnp.where(g16 + rows16 == 0, 1,
                         (slabv != prevv).astype(jnp.int32))
        fcv = plsc.cumsum(newv) + nf
        new_v[pl.ds(g16, 16)] = newv
        fc_v[pl.ds(g16, 16)] = fcv
        plsc.store_scatter(runs_v, [fcv - 1], slabv)
        return fcv[15]

    nruns = lax.fori_loop(0, NG, _pre, 0)

    def _fire(sl, slot):
        col = pl.multiple_of(sl * 128, 128)
        pltpu.async_copy(
            tblt_hbm.at[:, pl.ds(col, 128)], ring_v.at[slot], gsems.at[slot]
        )

    runs0 = runs_v[pl.ds(0, 16)]
    for r in range(FDEPTH):
        @pl.when(r < nruns)
        def _():
            _fire(runs0[r], r % KRING)

    def _zero(b, carry):
        for c in range(D // 16):
            big_v[b // OCHUNK, b % OCHUNK, pl.ds(D + c * 16, 16)] = zrow
        return carry

    lax.fori_loop(0, 2 * OCHUNK, _zero, 0)

    def _step(g, want_prev):
        g16 = g * 16
        idv = ids_v[pl.ds(g16, 16)]
        newv = new_v[pl.ds(g16, 16)]
        fcv = fc_v[pl.ds(g16, 16)]
        obank = (g // 4) % 2

        @pl.when(jnp.logical_and(g % 4 == 0, g >= 8))
        def _():
            pltpu.make_async_copy(
                big_v.at[obank], out_hbm.at[row_v.at[0]], osems.at[obank]
            ).wait()

        wp = want_prev
        for r in range(16):
            fci = fcv[r]
            slot = (fci - 1) % KRING

            want = jnp.minimum(fci + FDEPTH, nruns)
            fire_k = want - 1
            fslab = plsc.load_gather(runs_v, [jnp.full((16,), fire_k)])[0]

            @pl.when(want > wp)
            def _():
                _fire(fslab, fire_k % KRING)

            wp = want

            @pl.when(newv[r] == 1)
            def _():
                pltpu.make_async_copy(
                    tblt_hbm.at[:, pl.ds(0, 128)], ring_v.at[slot],
                    gsems.at[slot],
                ).wait()

            colv = jnp.full((16,), idv[r] & 127, jnp.int32)
            brow = (g % 4) * 16 + r
            for c in range(D // 16):
                vals = plsc.load_gather(ring_v.at[slot], [rows16 + c * 16, colv])
                big_v[obank, brow, pl.ds(c * 16, 16)] = vals

        @pl.when(g % 4 == 3)
        def _():
            pltpu.async_copy(
                big_v.at[obank],
                out_hbm.at[row_v.at[g // 4]],
                osems.at[obank],
            )

        return wp

    lax.fori_loop(0, NG, _step, jnp.minimum(FDEPTH, nruns))

    for obank in range(2):
        pltpu.make_async_copy(
            big_v.at[obank], out_hbm.at[row_v.at[0]], osems.at[obank]
        ).wait()


def kernel(entity_ids, entity_table):
    ids = entity_ids.astype(jnp.int32)
    pos = lax.iota(jnp.int32, B)
    key = ((ids >> 7) << 14) | pos
    key_s, ids_s = lax.sort((key, ids), num_keys=1, is_stable=False)
    order = key_s & (B - 1)
    rows = order.reshape(NW, NOCH, OCHUNK)
    return _lookup(ids_s.reshape(NW, BPW), rows, entity_table.T)
